# Initial kernel scaffold; baseline (speedup 1.0000x reference)
import jax
import jax.numpy as jnp
from jax.experimental import pallas as pl


def kernel(x, root_nodes, root_biases, tree_indices, nodes, biases, leaf_nodes):
    raise NotImplementedError("write your pallas kernel here")


# speedup vs baseline: 1380.4861x; 1380.4861x over previous
import functools

import jax
import jax.numpy as jnp
from jax import lax
from jax.experimental import pallas as pl
from jax.experimental.pallas import tpu as pltpu
from jax.experimental.pallas import tpu_sc as plsc

B, F, T, D = 16384, 256, 512, 8
NC, NS = 2, 16
NW = NC * NS
TGROUPS = 4
BGROUPS = NW // TGROUPS
TW = T // TGROUPS
BW = B // BGROUPS
CB = 32
NCHUNK = BW // CB
NLVL = D - 1
LVL = [TW * (2 ** i) for i in range(1, D)]


def _tree_body(*refs):
    x_hbm, rn_hbm, rb_hbm = refs[0:3]
    nd_hbm = refs[3:3 + NLVL]
    bs_hbm = refs[3 + NLVL:3 + 2 * NLVL]
    leaf_hbm = refs[3 + 2 * NLVL]
    out_hbm = refs[4 + 2 * NLVL]
    rn_v, rb_v = refs[5 + 2 * NLVL], refs[6 + 2 * NLVL]
    nd_v = refs[7 + 2 * NLVL:7 + 3 * NLVL]
    bs_v = refs[7 + 3 * NLVL:7 + 4 * NLVL]
    leaf_v = refs[7 + 4 * NLVL]
    x_v = refs[8 + 4 * NLVL]
    out_v = refs[9 + 4 * NLVL]

    c = lax.axis_index("c")
    s = lax.axis_index("s")
    wid = s * NC + c
    tg = wid % TGROUPS
    bg = wid // TGROUPS
    t0 = tg * TW
    row0 = bg * BW

    pltpu.sync_copy(rn_hbm.at[pl.ds(t0, TW)], rn_v)
    pltpu.sync_copy(rb_hbm.at[pl.ds(t0, TW)], rb_v)
    for i in range(NLVL):
        off = t0 * (2 ** (i + 1))
        pltpu.sync_copy(nd_hbm[i].at[pl.ds(off, LVL[i])], nd_v[i])
        pltpu.sync_copy(bs_hbm[i].at[pl.ds(off, LVL[i])], bs_v[i])
    pltpu.sync_copy(leaf_hbm.at[pl.ds(t0 * (2 ** D), TW * (2 ** D))], leaf_v)

    lane = lax.broadcasted_iota(jnp.int32, (16,), 0)

    def chunk_body(g, carry):
        r0 = row0 + g * CB
        pltpu.sync_copy(x_hbm.at[pl.ds(r0, CB)], x_v)

        def row_body(b, inner):
            brow = jnp.full((16,), b, dtype=jnp.int32)
            for tv in range(TW // 16):
                ltv = tv * 16 + lane
                rf = rn_v[pl.ds(tv * 16, 16)]
                rbv = rb_v[pl.ds(tv * 16, 16)]
                xv = plsc.load_gather(x_v, [brow, rf])
                prev = 2 * ltv + (xv >= rbv).astype(jnp.int32)
                for i in range(NLVL):
                    nf = plsc.load_gather(nd_v[i], [prev])
                    bv = plsc.load_gather(bs_v[i], [prev])
                    xv = plsc.load_gather(x_v, [brow, nf])
                    prev = 2 * prev + (xv >= bv).astype(jnp.int32)
                leaf = plsc.load_gather(leaf_v, [prev])
                out_v[b, pl.ds(tv * 16, 16)] = leaf
            return inner

        lax.fori_loop(0, CB, row_body, 0)
        pltpu.sync_copy(out_v, out_hbm.at[pl.ds(r0, CB), pl.ds(t0, TW)])
        return carry

    lax.fori_loop(0, NCHUNK, chunk_body, 0)


_tree_fn = pl.kernel(
    _tree_body,
    out_type=jax.ShapeDtypeStruct((B, T), jnp.float32),
    mesh=plsc.VectorSubcoreMesh(core_axis_name="c", subcore_axis_name="s"),
    compiler_params=pltpu.CompilerParams(
        use_tc_tiling_on_sc=False, needs_layout_passes=False),
    scratch_types=[
        pltpu.VMEM((TW,), jnp.int32),
        pltpu.VMEM((TW,), jnp.float32),
        *[pltpu.VMEM((LVL[i],), jnp.int32) for i in range(NLVL)],
        *[pltpu.VMEM((LVL[i],), jnp.float32) for i in range(NLVL)],
        pltpu.VMEM((TW * (2 ** D),), jnp.float32),
        pltpu.VMEM((CB, F), jnp.float32),
        pltpu.VMEM((CB, TW), jnp.float32),
    ],
)


def kernel(x, root_nodes, root_biases, tree_indices, nodes, biases, leaf_nodes):
    del tree_indices
    out = _tree_fn(x, root_nodes, root_biases, *nodes, *biases,
                   leaf_nodes.reshape(-1))
    return out.reshape(B, T, 1)
